# baseline (speedup 1.0000x reference)
import functools

import jax
import jax.numpy as jnp
from jax import lax
from jax.experimental import pallas as pl
from jax.experimental.pallas import tpu as pltpu
from jax.experimental.pallas import tpu_sc as plsc

_D = 64
_NT = 2
_OUT_W = _NT * _D
_NC = 2
_NS = 16
_NW = _NC * _NS
_L = 16
_BB = 128
_C_BUILD = 1024


def _build_table2_body(tok_ref, tabT_ref, out_ref):
    rows = tabT_ref[...].T
    out_ref[:, :_D] = rows + tok_ref[0:1, :]
    out_ref[:, _D:] = rows + tok_ref[1:2, :]


@functools.lru_cache(maxsize=None)
def _build_table2_call(num_rows):
    grid = pl.cdiv(num_rows, _C_BUILD)
    return pl.pallas_call(
        _build_table2_body,
        grid=(grid,),
        in_specs=[
            pl.BlockSpec((_NT, _D), lambda i: (0, 0)),
            pl.BlockSpec((_D, _C_BUILD), lambda i: (0, i)),
        ],
        out_specs=pl.BlockSpec((_C_BUILD, _OUT_W), lambda i: (i, 0)),
        out_shape=jax.ShapeDtypeStruct((num_rows, _OUT_W), jnp.float32),
    )


@functools.lru_cache(maxsize=None)
def _gather_call(n_t, n_b):
    assert n_b == _NW * _BB
    mesh = plsc.VectorSubcoreMesh(
        core_axis_name="c", subcore_axis_name="s", num_cores=_NC,
        num_subcores=_NS)

    def body(idxT_hbm, tab2_hbm, out_hbm,
             idxT_v, buf0, buf1, tb0, tb1, g0, g1, w0, w1, tsem):
        wid = lax.axis_index("s") * _NC + lax.axis_index("c")
        col0 = wid * _BB
        pltpu.sync_copy(idxT_hbm.at[:, pl.ds(col0, _BB)], idxT_v)
        bufs = (buf0, buf1)
        tbs = (tb0, tb1)
        gsems = (g0, g1)
        wsems = (w0, w1)
        rowidx = [lax.iota(jnp.int32, _L) + _L * g for g in range(_BB // _L)]

        pltpu.async_copy(
            tab2_hbm.at[idxT_v.at[0]], buf0.at[:, pl.ds(0, _OUT_W)], g0)

        def pair_body(i, carry):
            for k in range(2):
                t = 2 * i + k
                bufk, tbk = bufs[k], tbs[k]
                gk, wk = gsems[k], wsems[k]
                pltpu.make_async_copy(
                    tab2_hbm.at[pl.ds(0, _BB)],
                    bufk.at[:, pl.ds(0, _OUT_W)], gk).wait()

                @pl.when(t + 1 < n_t)
                def _():
                    pltpu.async_copy(
                        tab2_hbm.at[idxT_v.at[t + 1]],
                        bufs[1 - k].at[:, pl.ds(0, _OUT_W)],
                        gsems[1 - k])

                @pl.when(t >= 2)
                def _():
                    pltpu.make_async_copy(
                        tbk, out_hbm.at[pl.ds(0, _BB), pl.ds(0, _BB)],
                        wk).wait()

                @plsc.parallel_loop(0, _BB, unroll=2)
                def b_body(b):
                    colb = jnp.full((_L,), b, jnp.int32)
                    for g in range(_OUT_W // _L):
                        vals = bufk[b, pl.ds(g * _L, _L)]
                        plsc.store_scatter(tbk, [rowidx[g], colb], vals)
                pltpu.async_copy(
                    tbk,
                    out_hbm.at[pl.ds(t * _OUT_W, _OUT_W),
                               pl.ds(col0, _BB)],
                    wk)
            return carry

        lax.fori_loop(0, n_t // 2, pair_body, 0)
        pltpu.make_async_copy(
            tb0, out_hbm.at[pl.ds(0, _BB), pl.ds(0, _BB)], w0).wait()
        pltpu.make_async_copy(
            tb1, out_hbm.at[pl.ds(0, _BB), pl.ds(0, _BB)], w1).wait()

    return pl.kernel(
        body,
        out_type=jax.ShapeDtypeStruct((n_t * _OUT_W, n_b), jnp.float32),
        mesh=mesh,
        scratch_types=[
            pltpu.VMEM((n_t, _BB), jnp.int32),
            pltpu.VMEM((_BB, _OUT_W + 8), jnp.float32),
            pltpu.VMEM((_BB, _OUT_W + 8), jnp.float32),
            pltpu.VMEM((_OUT_W, _BB), jnp.float32),
            pltpu.VMEM((_OUT_W, _BB), jnp.float32),
            pltpu.SemaphoreType.DMA,
            pltpu.SemaphoreType.DMA,
            pltpu.SemaphoreType.DMA,
            pltpu.SemaphoreType.DMA,
            pltpu.SemaphoreType.DMA,
        ],
        compiler_params=pltpu.CompilerParams(needs_layout_passes=False),
    )


def kernel(actions, action_embed, learned_token):
    b, t = actions.shape
    actionsT = actions.T
    tableT = action_embed.T
    tok = learned_token.reshape(_NT, _D)
    table2 = _build_table2_call(action_embed.shape[0])(tok, tableT)
    out2 = _gather_call(t, b)(actionsT, table2)
    return out2.reshape(t, _NT, _D, b).transpose(3, 0, 1, 2)


# speedup vs baseline: 1.2004x; 1.2004x over previous
import functools

import jax
import jax.numpy as jnp
from jax import lax
from jax.experimental import pallas as pl
from jax.experimental.pallas import tpu as pltpu
from jax.experimental.pallas import tpu_sc as plsc

_D = 64
_NT = 2
_OUT_W = _NT * _D
_NC = 2
_NS = 16
_NW = _NC * _NS
_L = 16
_BB = 128
_C_BUILD = 1024


def _build_table2_body(tok_ref, tabT_ref, out_ref):
    rows = tabT_ref[...].T
    out_ref[:, :_D] = rows + tok_ref[0:1, :]
    out_ref[:, _D:] = rows + tok_ref[1:2, :]


@functools.lru_cache(maxsize=None)
def _build_table2_call(num_rows):
    grid = pl.cdiv(num_rows, _C_BUILD)
    return pl.pallas_call(
        _build_table2_body,
        grid=(grid,),
        in_specs=[
            pl.BlockSpec((_NT, _D), lambda i: (0, 0)),
            pl.BlockSpec((_D, _C_BUILD), lambda i: (0, i)),
        ],
        out_specs=pl.BlockSpec((_C_BUILD, _OUT_W), lambda i: (i, 0)),
        out_shape=jax.ShapeDtypeStruct((num_rows, _OUT_W), jnp.float32),
    )


@functools.lru_cache(maxsize=None)
def _gather_call(n_t, n_b):
    assert n_b == _NW * _BB
    mesh = plsc.VectorSubcoreMesh(
        core_axis_name="c", subcore_axis_name="s", num_cores=_NC,
        num_subcores=_NS)

    def body(idxT_hbm, tab2_hbm, out_hbm,
             idxT_v, buf0, buf1, tb0, tb1, g0, g1, w0, w1, tsem):
        wid = lax.axis_index("s") * _NC + lax.axis_index("c")
        col0 = wid * _BB
        pltpu.sync_copy(idxT_hbm.at[:, pl.ds(col0, _BB)], idxT_v)
        bufs = (buf0, buf1)
        tbs = (tb0, tb1)
        gsems = (g0, g1)
        wsems = (w0, w1)
        rowidx = [lax.iota(jnp.int32, _L) + _L * g for g in range(_BB // _L)]

        pltpu.async_copy(
            tab2_hbm.at[idxT_v.at[0]], buf0.at[:, pl.ds(0, _OUT_W)], g0)

        def pair_body(i, carry):
            for k in range(2):
                t = 2 * i + k
                bufk, tbk = bufs[k], tbs[k]
                gk, wk = gsems[k], wsems[k]
                pltpu.make_async_copy(
                    tab2_hbm.at[pl.ds(0, _BB)],
                    bufk.at[:, pl.ds(0, _OUT_W)], gk).wait()

                @pl.when(t + 1 < n_t)
                def _():
                    pltpu.async_copy(
                        tab2_hbm.at[idxT_v.at[t + 1]],
                        bufs[1 - k].at[:, pl.ds(0, _OUT_W)],
                        gsems[1 - k])

                @pl.when(t >= 2)
                def _():
                    pltpu.make_async_copy(
                        tbk, out_hbm.at[pl.ds(0, _BB), pl.ds(0, _BB)],
                        wk).wait()

                @plsc.parallel_loop(0, _D, unroll=2)
                def tp_body(i):
                    coli = jnp.full((_L,), i, jnp.int32)
                    for g in range(_BB // _L):
                        vals = plsc.load_gather(bufk, [rowidx[g], coli])
                        tbk[i, pl.ds(g * _L, _L)] = vals
                    for bb in range(2):
                        b = 2 * i + bb
                        colb = jnp.full((_L,), b, jnp.int32)
                        for g in range(4, _OUT_W // _L):
                            vals2 = bufk[b, pl.ds(g * _L, _L)]
                            plsc.store_scatter(tbk, [rowidx[g], colb], vals2)
                pltpu.async_copy(
                    tbk,
                    out_hbm.at[pl.ds(t * _OUT_W, _OUT_W),
                               pl.ds(col0, _BB)],
                    wk)
            return carry

        lax.fori_loop(0, n_t // 2, pair_body, 0)
        pltpu.make_async_copy(
            tb0, out_hbm.at[pl.ds(0, _BB), pl.ds(0, _BB)], w0).wait()
        pltpu.make_async_copy(
            tb1, out_hbm.at[pl.ds(0, _BB), pl.ds(0, _BB)], w1).wait()

    return pl.kernel(
        body,
        out_type=jax.ShapeDtypeStruct((n_t * _OUT_W, n_b), jnp.float32),
        mesh=mesh,
        scratch_types=[
            pltpu.VMEM((n_t, _BB), jnp.int32),
            pltpu.VMEM((_BB, _OUT_W + 8), jnp.float32),
            pltpu.VMEM((_BB, _OUT_W + 8), jnp.float32),
            pltpu.VMEM((_OUT_W, _BB), jnp.float32),
            pltpu.VMEM((_OUT_W, _BB), jnp.float32),
            pltpu.SemaphoreType.DMA,
            pltpu.SemaphoreType.DMA,
            pltpu.SemaphoreType.DMA,
            pltpu.SemaphoreType.DMA,
            pltpu.SemaphoreType.DMA,
        ],
        compiler_params=pltpu.CompilerParams(needs_layout_passes=False),
    )


def kernel(actions, action_embed, learned_token):
    b, t = actions.shape
    actionsT = actions.T
    tableT = action_embed.T
    tok = learned_token.reshape(_NT, _D)
    table2 = _build_table2_call(action_embed.shape[0])(tok, tableT)
    out2 = _gather_call(t, b)(actionsT, table2)
    return out2.reshape(t, _NT, _D, b).transpose(3, 0, 1, 2)
